# baseline (speedup 1.0000x reference)
import functools

import jax
import jax.numpy as jnp
from jax import lax
from jax.experimental import pallas as pl
from jax.experimental.pallas import tpu as pltpu
from jax.experimental.pallas import tpu_sc as plsc

D_MODEL = 128
HALF = 64
MAP_SIZE = 256
N_TYPES = 3
NTAB = N_TYPES * MAP_SIZE
ROWS = 4096 * 64

_NC, _NS = 2, 16
_NW = _NC * _NS
_RPW = ROWS // _NW
_CH = 128
_WPR = HALF // 2
_CH2 = 2 * _CH
_NCH = _RPW // _CH
_NBUF = 2
_NGRP = _NCH // _NBUF


def _sc_gather(table, idxw):
    mesh = plsc.VectorSubcoreMesh(core_axis_name="c", subcore_axis_name="s")

    def vtake(v, idxvec):
        return lax.gather(
            v,
            idxvec[:, None],
            lax.GatherDimensionNumbers(
                offset_dims=(), collapsed_slice_dims=(0,), start_index_map=(0,)
            ),
            (1,),
            mode=lax.GatherScatterMode.PROMISE_IN_BOUNDS,
        )

    @functools.partial(
        pl.kernel,
        out_type=jax.ShapeDtypeStruct((ROWS * HALF,), jnp.int32),
        mesh=mesh,
        scratch_types=(
            [pltpu.VMEM((2 * NTAB * _WPR,), jnp.int32)]
            + [pltpu.VMEM((_CH2,), jnp.int32) for _ in range(_NBUF)]
            + [pltpu.VMEM((_CH * HALF,), jnp.int32) for _ in range(_NBUF)]
            + [pltpu.SemaphoreType.DMA for _ in range(2 * _NBUF)]
        ),
        compiler_params=pltpu.CompilerParams(needs_layout_passes=False),
    )
    def k(tab_hbm, idx_hbm, out_hbm, tab_v, *bufs):
        ibufs = bufs[:_NBUF]
        obufs = bufs[_NBUF:2 * _NBUF]
        sis = bufs[2 * _NBUF:3 * _NBUF]
        sos = bufs[3 * _NBUF:4 * _NBUF]

        wid = lax.axis_index("s") * _NC + lax.axis_index("c")
        base = wid * _RPW

        pltpu.sync_copy(tab_hbm, tab_v)
        iota16 = lax.iota(jnp.int32, 16)
        svecs = [iota16 + k * 16 for k in range(_WPR // 16)]
        zero16 = iota16 * 0
        bidx = [zero16 + j for j in range(16)]

        def idx_src(c):
            return idx_hbm.at[pl.ds(base * 2 + c * _CH2, _CH2)]

        def out_dst(c):
            return out_hbm.at[pl.ds((base + c * _CH) * HALF, _CH * HALF)]

        for b in range(_NBUF):
            pltpu.async_copy(idx_src(b), ibufs[b], sis[b])

        def group(g, carry):
            for b in range(_NBUF):
                c = g * _NBUF + b
                pltpu.make_async_copy(idx_src(c), ibufs[b], sis[b]).wait()

                @pl.when(g > 0)
                def _():
                    pltpu.make_async_copy(obufs[b], out_dst(c), sos[b]).wait()

                for grp in range(_CH // 8):
                    iv = ibufs[b][pl.ds(grp * 16, 16)]
                    for m in range(8):
                        r = grp * 8 + m
                        i1 = vtake(iv, bidx[2 * m])
                        i2 = vtake(iv, bidx[2 * m + 1])
                        for k in range(_WPR // 16):
                            sv = svecs[k]
                            v = plsc.load_gather(tab_v, [i1 + sv])
                            plsc.store_scatter(obufs[b], [sv + r * HALF], v)
                            w = plsc.load_gather(tab_v, [i2 + sv])
                            plsc.store_scatter(obufs[b], [sv + (r * HALF + _WPR)], w)

                pltpu.async_copy(obufs[b], out_dst(c), sos[b])

                @pl.when(c + _NBUF < _NCH)
                def _():
                    pltpu.async_copy(idx_src(c + _NBUF), ibufs[b], sis[b])

            return carry

        lax.fori_loop(0, _NGRP, group, 0)

        for b in range(_NBUF):
            pltpu.make_async_copy(obufs[b], out_dst(_NCH - _NBUF + b), sos[b]).wait()

    return k(table, idxw)


def _tc_combine(g2d, s2d, wt, b2d):
    blk = 2048

    def body(g_ref, s_ref, w_ref, b_ref, o_ref):
        fc = jnp.dot(s_ref[...], w_ref[...], preferred_element_type=jnp.float32)
        fc = jnp.maximum(fc + b_ref[...], 0.0)
        o_ref[...] = g_ref[...].astype(jnp.float32) + fc

    return pl.pallas_call(
        body,
        grid=(ROWS // blk,),
        in_specs=[
            pl.BlockSpec((blk, D_MODEL), lambda i: (i, 0)),
            pl.BlockSpec((blk, 16), lambda i: (i, 0)),
            pl.BlockSpec((16, D_MODEL), lambda i: (0, 0)),
            pl.BlockSpec((1, D_MODEL), lambda i: (0, 0)),
        ],
        out_specs=pl.BlockSpec((blk, D_MODEL), lambda i: (i, 0)),
        out_shape=jax.ShapeDtypeStruct((ROWS, D_MODEL), jnp.float32),
    )(g2d, s2d, wt, b2d)


def kernel(typ, pos, scalar, type_table, W, b, pe):
    tw = pe[:HALF, 0, :].T
    th = pe[HALF:, :, 0].T
    c1 = type_table[:, None, :HALF] + tw[None]
    c2 = type_table[:, None, HALF:] + th[None]
    table = jnp.concatenate(
        [c1.reshape(NTAB, HALF), c2.reshape(NTAB, HALF)], axis=0
    )
    tpk = table.astype(jnp.bfloat16).reshape(2 * NTAB, _WPR, 2)
    tab_i32 = lax.bitcast_convert_type(tpk, jnp.int32).reshape(2 * NTAB * _WPR)
    i1 = typ * MAP_SIZE + pos[..., 1]
    i2 = NTAB + typ * MAP_SIZE + pos[..., 0]
    idxw = (jnp.stack([i1, i2], axis=-1) * _WPR).reshape(2 * ROWS)

    g = _sc_gather(tab_i32, idxw)
    g_bf = lax.bitcast_convert_type(
        g.reshape(ROWS, HALF), jnp.bfloat16
    ).reshape(ROWS, D_MODEL)
    out = _tc_combine(
        g_bf,
        scalar.reshape(ROWS, 16),
        W.T,
        b.reshape(1, D_MODEL),
    )
    return out.reshape(typ.shape[0], typ.shape[1], D_MODEL)


# speedup vs baseline: 1.8290x; 1.8290x over previous
import functools

import jax
import jax.numpy as jnp
from jax import lax
from jax.experimental import pallas as pl
from jax.experimental.pallas import tpu as pltpu
from jax.experimental.pallas import tpu_sc as plsc

D_MODEL = 128
HALF = 64
MAP_SIZE = 256
N_TYPES = 3
NTAB = N_TYPES * MAP_SIZE
ROWS = 4096 * 64

_NC, _NS = 2, 16
_NW = _NC * _NS
_RPW = ROWS // _NW
_CH = 128
_WPR = HALF // 2
_CH2 = 2 * _CH
_NCH = _RPW // _CH
_NBUF = 2
_NGRP = _NCH // _NBUF


def _sc_gather(table, idxw):
    mesh = plsc.VectorSubcoreMesh(core_axis_name="c", subcore_axis_name="s")

    def vtake(v, idxvec):
        return lax.gather(
            v,
            idxvec[:, None],
            lax.GatherDimensionNumbers(
                offset_dims=(), collapsed_slice_dims=(0,), start_index_map=(0,)
            ),
            (1,),
            mode=lax.GatherScatterMode.PROMISE_IN_BOUNDS,
        )

    @functools.partial(
        pl.kernel,
        out_type=jax.ShapeDtypeStruct((ROWS * HALF,), jnp.int32),
        mesh=mesh,
        scratch_types=(
            [pltpu.VMEM((2 * NTAB * _WPR,), jnp.int32)]
            + [pltpu.VMEM((_CH2,), jnp.int32) for _ in range(_NBUF)]
            + [pltpu.VMEM((_CH * HALF,), jnp.int32) for _ in range(_NBUF)]
            + [pltpu.SemaphoreType.DMA for _ in range(2 * _NBUF)]
        ),
        compiler_params=pltpu.CompilerParams(needs_layout_passes=False),
    )
    def k(tab_hbm, idx_hbm, out_hbm, tab_v, *bufs):
        ibufs = bufs[:_NBUF]
        obufs = bufs[_NBUF:2 * _NBUF]
        sis = bufs[2 * _NBUF:3 * _NBUF]
        sos = bufs[3 * _NBUF:4 * _NBUF]

        wid = lax.axis_index("s") * _NC + lax.axis_index("c")
        base = wid * _RPW

        pltpu.sync_copy(tab_hbm, tab_v)
        iota16 = lax.iota(jnp.int32, 16)
        svecs = [iota16 + k * 16 for k in range(_WPR // 16)]
        zero16 = iota16 * 0
        bidx = [zero16 + j for j in range(16)]

        def idx_src(c):
            return idx_hbm.at[pl.ds(base * 2 + c * _CH2, _CH2)]

        def out_dst(c):
            return out_hbm.at[pl.ds((base + c * _CH) * HALF, _CH * HALF)]

        for b in range(_NBUF):
            pltpu.async_copy(idx_src(b), ibufs[b], sis[b])

        def group(g, carry):
            for b in range(_NBUF):
                c = g * _NBUF + b
                pltpu.make_async_copy(idx_src(c), ibufs[b], sis[b]).wait()

                @pl.when(g > 0)
                def _():
                    pltpu.make_async_copy(obufs[b], out_dst(c), sos[b]).wait()

                for grp in range(_CH // 8):
                    iv = ibufs[b][pl.ds(grp * 16, 16)]
                    for m in range(8):
                        r = grp * 8 + m
                        i1 = vtake(iv, bidx[2 * m])
                        i2 = vtake(iv, bidx[2 * m + 1])
                        for k in range(_WPR // 16):
                            sv = svecs[k]
                            v = plsc.load_gather(tab_v, [i1 + sv])
                            plsc.store_scatter(obufs[b], [sv + r * HALF], v)
                            w = plsc.load_gather(tab_v, [i2 + sv])
                            plsc.store_scatter(obufs[b], [sv + (r * HALF + _WPR)], w)

                pltpu.async_copy(obufs[b], out_dst(c), sos[b])

                @pl.when(c + _NBUF < _NCH)
                def _():
                    pltpu.async_copy(idx_src(c + _NBUF), ibufs[b], sis[b])

            return carry

        lax.fori_loop(0, _NGRP, group, 0)

        for b in range(_NBUF):
            pltpu.make_async_copy(obufs[b], out_dst(_NCH - _NBUF + b), sos[b]).wait()

    return k(table, idxw)


def _tc_combine(g2d, s2d, wt, b2d):
    blk = 2048

    def body(g_ref, s_ref, w_ref, b_ref, pea_ref, poa_ref, peb_ref, pob_ref, o_ref):
        fc = jnp.dot(s_ref[...], w_ref[...], preferred_element_type=jnp.float32)
        fc = jnp.maximum(fc + b_ref[...], 0.0)
        gi = g_ref[...]
        lo = lax.bitcast_convert_type(gi << 16, jnp.float32).astype(jnp.bfloat16)
        hi = lax.bitcast_convert_type(
            gi & jnp.int32(-65536), jnp.float32
        ).astype(jnp.bfloat16)
        ra = (
            jnp.dot(lo, pea_ref[...], preferred_element_type=jnp.float32)
            + jnp.dot(hi, poa_ref[...], preferred_element_type=jnp.float32)
        ).reshape(blk // 2, 1, D_MODEL)
        rb = (
            jnp.dot(lo, peb_ref[...], preferred_element_type=jnp.float32)
            + jnp.dot(hi, pob_ref[...], preferred_element_type=jnp.float32)
        ).reshape(blk // 2, 1, D_MODEL)
        gr = jnp.concatenate([ra, rb], axis=1).reshape(blk, D_MODEL)
        o_ref[...] = gr + fc

    qd = D_MODEL // 2
    cc = jnp.arange(D_MODEL)
    ch = (2 * (cc % qd))[:, None]
    tgt = jnp.arange(D_MODEL)[None, :]
    pea = ((cc[:, None] < qd) & (tgt == ch)).astype(jnp.bfloat16)
    poa = ((cc[:, None] < qd) & (tgt == ch + 1)).astype(jnp.bfloat16)
    peb = ((cc[:, None] >= qd) & (tgt == ch)).astype(jnp.bfloat16)
    pob = ((cc[:, None] >= qd) & (tgt == ch + 1)).astype(jnp.bfloat16)

    return pl.pallas_call(
        body,
        grid=(ROWS // blk,),
        in_specs=[
            pl.BlockSpec((blk // 2, D_MODEL), lambda i: (i, 0)),
            pl.BlockSpec((blk, 16), lambda i: (i, 0)),
            pl.BlockSpec((16, D_MODEL), lambda i: (0, 0)),
            pl.BlockSpec((1, D_MODEL), lambda i: (0, 0)),
            pl.BlockSpec((D_MODEL, D_MODEL), lambda i: (0, 0)),
            pl.BlockSpec((D_MODEL, D_MODEL), lambda i: (0, 0)),
            pl.BlockSpec((D_MODEL, D_MODEL), lambda i: (0, 0)),
            pl.BlockSpec((D_MODEL, D_MODEL), lambda i: (0, 0)),
        ],
        out_specs=pl.BlockSpec((blk, D_MODEL), lambda i: (i, 0)),
        out_shape=jax.ShapeDtypeStruct((ROWS, D_MODEL), jnp.float32),
    )(g2d, s2d, wt, b2d, pea, poa, peb, pob)


def kernel(typ, pos, scalar, type_table, W, b, pe):
    tw = pe[:HALF, 0, :].T
    th = pe[HALF:, :, 0].T
    c1 = type_table[:, None, :HALF] + tw[None]
    c2 = type_table[:, None, HALF:] + th[None]
    table = jnp.concatenate(
        [c1.reshape(NTAB, HALF), c2.reshape(NTAB, HALF)], axis=0
    )
    tpk = table.astype(jnp.bfloat16).reshape(2 * NTAB, _WPR, 2)
    tab_i32 = lax.bitcast_convert_type(tpk, jnp.int32).reshape(2 * NTAB * _WPR)
    i1 = typ * MAP_SIZE + pos[..., 1]
    i2 = NTAB + typ * MAP_SIZE + pos[..., 0]
    idxw = (jnp.stack([i1, i2], axis=-1) * _WPR).reshape(2 * ROWS)

    g = _sc_gather(tab_i32, idxw)
    out = _tc_combine(
        g.reshape(ROWS // 2, D_MODEL),
        scalar.reshape(ROWS, 16),
        W.T,
        b.reshape(1, D_MODEL),
    )
    return out.reshape(typ.shape[0], typ.shape[1], D_MODEL)
